# baseline (speedup 1.0000x reference)
import jax
import jax.numpy as jnp
from jax import lax
from jax.experimental import pallas as pl
from jax.experimental.pallas import tpu as pltpu
from jax.experimental.pallas import tpu_sc as plsc

N_ROWS = 16384
N_COLS = 64
B = N_ROWS * N_COLS
NW = 32
CHUNK = B // NW
L = 16
NVEC = CHUNK // L


def _gather_body(src_hbm, idx_hbm, out_hbm, idx_v, out_v, sem):
    c = lax.axis_index("c")
    s = lax.axis_index("s")
    wid = s * 2 + c
    base = wid * CHUNK

    pltpu.sync_copy(idx_hbm.at[pl.ds(base, CHUNK)], idx_v)

    lane = lax.iota(jnp.int32, 16)

    def body(i, carry):
        off = lane + lax.rem(i, 4) * L
        v = idx_v[pl.ds(i * L, L)]
        idx_v[pl.ds(i * L, L)] = v * N_COLS + off
        return carry

    lax.fori_loop(0, NVEC, body, 0)

    pltpu.async_copy(src_hbm.at[idx_v], out_v, sem).wait()

    pltpu.sync_copy(out_v, out_hbm.at[pl.ds(base, CHUNK)])


def kernel(source, source_idx_2d):
    src_flat = source.reshape(-1)
    idx_flat = source_idx_2d.reshape(-1).astype(jnp.int32)
    mesh = plsc.VectorSubcoreMesh(core_axis_name="c", subcore_axis_name="s")
    out = pl.kernel(
        _gather_body,
        out_type=jax.ShapeDtypeStruct((B,), jnp.float32),
        mesh=mesh,
        scratch_types=[
            pltpu.VMEM((CHUNK,), jnp.int32),
            pltpu.VMEM((CHUNK,), jnp.float32),
            pltpu.SemaphoreType.DMA,
        ],
    )(src_flat, idx_flat)
    return out.reshape(N_ROWS, N_COLS)


# speedup vs baseline: 1.0044x; 1.0044x over previous
import jax
import jax.numpy as jnp
from jax import lax
from jax.experimental import pallas as pl
from jax.experimental.pallas import tpu as pltpu
from jax.experimental.pallas import tpu_sc as plsc

N_ROWS = 16384
N_COLS = 64
B = N_ROWS * N_COLS
NW = 32
W_ELEMS = B // NW
ROWS_PER_W = N_ROWS // NW
L = 16
QS = N_COLS // L


def _gather_body(src_hbm, idx_hbm, out_hbm, a_v, flat_v, g_v, sem):
    c = lax.axis_index("c")
    s = lax.axis_index("s")
    wid = s * 2 + c
    rb = wid * ROWS_PER_W

    a_i32 = a_v.bitcast(jnp.int32)

    pltpu.sync_copy(idx_hbm.at[pl.ds(rb, ROWS_PER_W)], a_i32)

    lane = lax.iota(jnp.int32, L)

    def arith(r, carry):
        for q in range(QS):
            v = a_i32[r, pl.ds(q * L, L)]
            flat_v[pl.ds(r * N_COLS + q * L, L)] = v * N_COLS + q * L + lane
        return carry

    lax.fori_loop(0, ROWS_PER_W, arith, 0)

    pltpu.async_copy(src_hbm.at[flat_v], g_v, sem).wait()

    def repack(r, carry):
        for q in range(QS):
            a_v[r, pl.ds(q * L, L)] = g_v[pl.ds(r * N_COLS + q * L, L)]
        return carry

    lax.fori_loop(0, ROWS_PER_W, repack, 0)

    pltpu.sync_copy(a_v, out_hbm.at[pl.ds(rb, ROWS_PER_W)])


def kernel(source, source_idx_2d):
    src_flat = source.reshape(-1)
    idx = source_idx_2d.astype(jnp.int32)
    mesh = plsc.VectorSubcoreMesh(core_axis_name="c", subcore_axis_name="s")
    return pl.kernel(
        _gather_body,
        out_type=jax.ShapeDtypeStruct((N_ROWS, N_COLS), jnp.float32),
        mesh=mesh,
        scratch_types=[
            pltpu.VMEM((ROWS_PER_W, N_COLS), jnp.float32),
            pltpu.VMEM((W_ELEMS,), jnp.int32),
            pltpu.VMEM((W_ELEMS,), jnp.float32),
            pltpu.SemaphoreType.DMA,
        ],
    )(src_flat, idx)
